# Initial kernel scaffold; baseline (speedup 1.0000x reference)
import jax
import jax.numpy as jnp
from jax.experimental import pallas as pl


def kernel(user_indices, item_indices, user_table, item_table, W, b):
    raise NotImplementedError("write your pallas kernel here")


# speedup vs baseline: 7.3061x; 7.3061x over previous
import functools

import jax
import jax.numpy as jnp
from jax import lax
from jax.experimental import pallas as pl
from jax.experimental.pallas import tpu as pltpu
from jax.experimental.pallas import tpu_sc as plsc

_B = 16384
_D = 128
_NC = 2
_NS = 16
_NW = _NC * _NS
_BPW = _B // _NW
_CH = 128
_NCH = _BPW // _CH

_EO = 300
_EOP = 384
_RB = 1024
_NRB = _B // _RB


def _gather_body(utab, itab, uidx, iidx, out_u, out_v,
                 uidx_v, iidx_v, buf0, buf1, g0, g1, o0, o1):
    wid = lax.axis_index("s") * _NC + lax.axis_index("c")
    base = wid * _BPW
    pltpu.sync_copy(uidx.at[wid], uidx_v)
    pltpu.sync_copy(iidx.at[wid], iidx_v)
    bufs = (buf0, buf1)
    gsem = (g0, g1)
    osem = (o0, o1)
    jobs = ([(utab, uidx_v, out_u, j) for j in range(_NCH)]
            + [(itab, iidx_v, out_v, j) for j in range(_NCH)])
    n = len(jobs)
    gathers = [None] * n
    outs = [None] * n
    for k in range(n):
        tab, idxv, _, j = jobs[k]
        if k >= 2:
            outs[k - 2].wait()
        gathers[k] = pltpu.async_copy(tab.at[idxv.at[j]], bufs[k % 2],
                                      gsem[k % 2])
        if k >= 1:
            _, _, pout, pj = jobs[k - 1]
            gathers[k - 1].wait()
            outs[k - 1] = pltpu.async_copy(
                bufs[(k - 1) % 2], pout.at[pl.ds(base + pj * _CH, _CH)],
                osem[(k - 1) % 2])
    _, _, lout, lj = jobs[n - 1]
    gathers[n - 1].wait()
    outs[n - 1] = pltpu.async_copy(
        bufs[(n - 1) % 2], lout.at[pl.ds(base + lj * _CH, _CH)],
        osem[(n - 1) % 2])
    outs[n - 2].wait()
    outs[n - 1].wait()


@functools.cache
def _make_gather():
    return functools.partial(
        pl.kernel,
        mesh=plsc.VectorSubcoreMesh(core_axis_name="c", subcore_axis_name="s"),
        out_type=[jax.ShapeDtypeStruct((_B, _D), jnp.float32),
                  jax.ShapeDtypeStruct((_B, _D), jnp.float32)],
        scratch_types=[
            pltpu.VMEM((_NCH, _CH), jnp.int32),
            pltpu.VMEM((_NCH, _CH), jnp.int32),
            pltpu.VMEM((_CH, _D), jnp.float32),
            pltpu.VMEM((_CH, _D), jnp.float32),
            pltpu.SemaphoreType.DMA,
            pltpu.SemaphoreType.DMA,
            pltpu.SemaphoreType.DMA,
            pltpu.SemaphoreType.DMA,
        ],
    )(_gather_body)


def _dense_body(ue_ref, ve_ref, w_ref, b_ref, out_ref):
    u = jnp.dot(ue_ref[...], w_ref[...],
                preferred_element_type=jnp.float32) + b_ref[...]
    v = jnp.dot(ve_ref[...], w_ref[...],
                preferred_element_type=jnp.float32) + b_ref[...]
    u = jnp.maximum(u, 0.0)
    v = jnp.maximum(v, 0.0)
    num = jnp.sum(u * v, axis=1)
    den = jnp.sqrt(jnp.sum(u * u, axis=1) * jnp.sum(v * v, axis=1))
    out_ref[...] = (num / jnp.maximum(den, 1e-8))[None, None, :]


_dense = pl.pallas_call(
    _dense_body,
    grid=(_NRB,),
    in_specs=[
        pl.BlockSpec((_RB, _D), lambda i: (i, 0)),
        pl.BlockSpec((_RB, _D), lambda i: (i, 0)),
        pl.BlockSpec((_D, _EOP), lambda i: (0, 0)),
        pl.BlockSpec((1, _EOP), lambda i: (0, 0)),
    ],
    out_specs=pl.BlockSpec((1, 1, _RB), lambda i: (i, 0, 0)),
    out_shape=jax.ShapeDtypeStruct((_NRB, 1, _RB), jnp.float32),
    compiler_params=pltpu.CompilerParams(
        dimension_semantics=("arbitrary",)),
)


def kernel(user_indices, item_indices, user_table, item_table, W, b):
    uidx = user_indices.astype(jnp.int32).reshape(_NW, _NCH, _CH)
    iidx = item_indices.astype(jnp.int32).reshape(_NW, _NCH, _CH)
    ue, ve = _make_gather()(user_table, item_table, uidx, iidx)
    wp = jnp.pad(W, ((0, 0), (0, _EOP - _EO)))
    bp = jnp.pad(b, (0, _EOP - _EO)).reshape(1, _EOP)
    scores = _dense(ue, ve, wp, bp)
    return scores.reshape(_B)


# speedup vs baseline: 7.6650x; 1.0491x over previous
import functools

import jax
import jax.numpy as jnp
from jax import lax
from jax.experimental import pallas as pl
from jax.experimental.pallas import tpu as pltpu
from jax.experimental.pallas import tpu_sc as plsc

_B = 16384
_D = 128
_NC = 2
_NS = 16
_NW = _NC * _NS
_BPW = _B // _NW
_CH = 128
_NCH = _BPW // _CH

_EO = 300
_EOP = 384
_RB = 2048
_NRB = _B // _RB


def _gather_body(utab, itab, uidx, iidx, out_u, out_v,
                 uidx_v, iidx_v, buf0, buf1, g0, g1, o0, o1):
    wid = lax.axis_index("s") * _NC + lax.axis_index("c")
    base = wid * _BPW
    pltpu.sync_copy(uidx.at[wid], uidx_v)
    pltpu.sync_copy(iidx.at[wid], iidx_v)
    bufs = (buf0, buf1)
    gsem = (g0, g1)
    osem = (o0, o1)
    jobs = ([(utab, uidx_v, out_u, j) for j in range(_NCH)]
            + [(itab, iidx_v, out_v, j) for j in range(_NCH)])
    n = len(jobs)
    gathers = [None] * n
    outs = [None] * n
    for k in range(n):
        tab, idxv, _, j = jobs[k]
        if k >= 2:
            outs[k - 2].wait()
        gathers[k] = pltpu.async_copy(tab.at[idxv.at[j]], bufs[k % 2],
                                      gsem[k % 2])
        if k >= 1:
            _, _, pout, pj = jobs[k - 1]
            gathers[k - 1].wait()
            outs[k - 1] = pltpu.async_copy(
                bufs[(k - 1) % 2], pout.at[pl.ds(base + pj * _CH, _CH)],
                osem[(k - 1) % 2])
    _, _, lout, lj = jobs[n - 1]
    gathers[n - 1].wait()
    outs[n - 1] = pltpu.async_copy(
        bufs[(n - 1) % 2], lout.at[pl.ds(base + lj * _CH, _CH)],
        osem[(n - 1) % 2])
    outs[n - 2].wait()
    outs[n - 1].wait()


@functools.cache
def _make_gather():
    return functools.partial(
        pl.kernel,
        mesh=plsc.VectorSubcoreMesh(core_axis_name="c", subcore_axis_name="s"),
        out_type=[jax.ShapeDtypeStruct((_B, _D), jnp.float32),
                  jax.ShapeDtypeStruct((_B, _D), jnp.float32)],
        scratch_types=[
            pltpu.VMEM((_NCH, _CH), jnp.int32),
            pltpu.VMEM((_NCH, _CH), jnp.int32),
            pltpu.VMEM((_CH, _D), jnp.float32),
            pltpu.VMEM((_CH, _D), jnp.float32),
            pltpu.SemaphoreType.DMA,
            pltpu.SemaphoreType.DMA,
            pltpu.SemaphoreType.DMA,
            pltpu.SemaphoreType.DMA,
        ],
    )(_gather_body)


def _dense_body(ue_ref, ve_ref, w_ref, b_ref, out_ref):
    u = jnp.dot(ue_ref[...], w_ref[...],
                preferred_element_type=jnp.float32) + b_ref[...]
    v = jnp.dot(ve_ref[...], w_ref[...],
                preferred_element_type=jnp.float32) + b_ref[...]
    u = jnp.maximum(u, 0.0)
    v = jnp.maximum(v, 0.0)
    num = jnp.sum(u * v, axis=1, keepdims=True)
    den = jnp.sqrt(jnp.sum(u * u, axis=1, keepdims=True)
                   * jnp.sum(v * v, axis=1, keepdims=True))
    out_ref[...] = num / jnp.maximum(den, 1e-8)


_dense = pl.pallas_call(
    _dense_body,
    grid=(_NRB,),
    in_specs=[
        pl.BlockSpec((_RB, _D), lambda i: (i, 0)),
        pl.BlockSpec((_RB, _D), lambda i: (i, 0)),
        pl.BlockSpec((_D, _EOP), lambda i: (0, 0)),
        pl.BlockSpec((1, _EOP), lambda i: (0, 0)),
    ],
    out_specs=pl.BlockSpec((_RB, 1), lambda i: (i, 0)),
    out_shape=jax.ShapeDtypeStruct((_B, 1), jnp.float32),
    compiler_params=pltpu.CompilerParams(
        dimension_semantics=("arbitrary",)),
)


def kernel(user_indices, item_indices, user_table, item_table, W, b):
    uidx = user_indices.astype(jnp.int32).reshape(_NW, _NCH, _CH)
    iidx = item_indices.astype(jnp.int32).reshape(_NW, _NCH, _CH)
    ue, ve = _make_gather()(user_table, item_table, uidx, iidx)
    wp = jnp.pad(W, ((0, 0), (0, _EOP - _EO)))
    bp = jnp.pad(b, (0, _EOP - _EO)).reshape(1, _EOP)
    scores = _dense(ue, ve, wp, bp)
    return scores.reshape(_B)


# speedup vs baseline: 8.3714x; 1.0922x over previous
import functools

import jax
import jax.numpy as jnp
from jax import lax
from jax.experimental import pallas as pl
from jax.experimental.pallas import tpu as pltpu
from jax.experimental.pallas import tpu_sc as plsc

_B = 16384
_D = 128
_NC = 2
_NS = 16
_NW = _NC * _NS
_BPW = _B // _NW
_CH = 128
_NCH = _BPW // _CH

_EO = 300
_EOP = 384
_RB = 2048
_NRB = _B // _RB


def _gather_body(utab, itab, uidx, iidx, out_u, out_v,
                 uidx_v, iidx_v, buf0, buf1, g0, g1, o0, o1):
    wid = lax.axis_index("s") * _NC + lax.axis_index("c")
    base = wid * _BPW
    pltpu.sync_copy(uidx.at[wid], uidx_v)
    pltpu.sync_copy(iidx.at[wid], iidx_v)
    bufs = (buf0, buf1)
    gsem = (g0, g1)
    osem = (o0, o1)
    jobs = ([(utab, uidx_v, out_u, j) for j in range(_NCH)]
            + [(itab, iidx_v, out_v, j) for j in range(_NCH)])
    n = len(jobs)
    gathers = [None] * n
    outs = [None] * n
    for k in range(n):
        tab, idxv, _, j = jobs[k]
        if k >= 2:
            outs[k - 2].wait()
        gathers[k] = pltpu.async_copy(tab.at[idxv.at[j]], bufs[k % 2],
                                      gsem[k % 2])
        if k >= 1:
            _, _, pout, pj = jobs[k - 1]
            gathers[k - 1].wait()
            outs[k - 1] = pltpu.async_copy(
                bufs[(k - 1) % 2], pout.at[pl.ds(base + pj * _CH, _CH)],
                osem[(k - 1) % 2])
    _, _, lout, lj = jobs[n - 1]
    gathers[n - 1].wait()
    outs[n - 1] = pltpu.async_copy(
        bufs[(n - 1) % 2], lout.at[pl.ds(base + lj * _CH, _CH)],
        osem[(n - 1) % 2])
    outs[n - 2].wait()
    outs[n - 1].wait()


@functools.cache
def _make_gather():
    return functools.partial(
        pl.kernel,
        mesh=plsc.VectorSubcoreMesh(core_axis_name="c", subcore_axis_name="s"),
        out_type=[jax.ShapeDtypeStruct((_B, _D), jnp.float32),
                  jax.ShapeDtypeStruct((_B, _D), jnp.float32)],
        scratch_types=[
            pltpu.VMEM((_NCH, _CH), jnp.int32),
            pltpu.VMEM((_NCH, _CH), jnp.int32),
            pltpu.VMEM((_CH, _D), jnp.float32),
            pltpu.VMEM((_CH, _D), jnp.float32),
            pltpu.SemaphoreType.DMA,
            pltpu.SemaphoreType.DMA,
            pltpu.SemaphoreType.DMA,
            pltpu.SemaphoreType.DMA,
        ],
    )(_gather_body)


_TCH = 256


def _dense_body(ue_ref, ve_ref, w_ref, b_ref, eye_ref, out_ref):
    u = jnp.dot(ue_ref[...], w_ref[...],
                preferred_element_type=jnp.float32) + b_ref[...]
    v = jnp.dot(ve_ref[...], w_ref[...],
                preferred_element_type=jnp.float32) + b_ref[...]
    u = jnp.maximum(u, 0.0)
    v = jnp.maximum(v, 0.0)
    num = jnp.sum(u * v, axis=1, keepdims=True)
    den = jnp.sqrt(jnp.sum(u * u, axis=1, keepdims=True)
                   * jnp.sum(v * v, axis=1, keepdims=True))
    s_col = num / jnp.maximum(den, 1e-8)
    eye = eye_ref[...]
    rows = [
        jax.lax.dot_general(
            s_col[i * _TCH:(i + 1) * _TCH, :], eye,
            (((0,), (0,)), ((), ())),
            preferred_element_type=jnp.float32)
        for i in range(_RB // _TCH)
    ]
    out_ref[...] = jnp.concatenate(rows, axis=1)[None]


_dense = pl.pallas_call(
    _dense_body,
    grid=(_NRB,),
    in_specs=[
        pl.BlockSpec((_RB, _D), lambda i: (i, 0)),
        pl.BlockSpec((_RB, _D), lambda i: (i, 0)),
        pl.BlockSpec((_D, _EOP), lambda i: (0, 0)),
        pl.BlockSpec((1, _EOP), lambda i: (0, 0)),
        pl.BlockSpec((_TCH, _TCH), lambda i: (0, 0)),
    ],
    out_specs=pl.BlockSpec((1, 1, _RB), lambda i: (i, 0, 0)),
    out_shape=jax.ShapeDtypeStruct((_NRB, 1, _RB), jnp.float32),
    compiler_params=pltpu.CompilerParams(
        dimension_semantics=("arbitrary",)),
)


def kernel(user_indices, item_indices, user_table, item_table, W, b):
    uidx = user_indices.astype(jnp.int32).reshape(_NW, _NCH, _CH)
    iidx = item_indices.astype(jnp.int32).reshape(_NW, _NCH, _CH)
    ue, ve = _make_gather()(user_table, item_table, uidx, iidx)
    wp = jnp.pad(W, ((0, 0), (0, _EOP - _EO)))
    bp = jnp.pad(b, (0, _EOP - _EO)).reshape(1, _EOP)
    eye = jnp.eye(_TCH, dtype=jnp.float32)
    scores = _dense(ue, ve, wp, bp, eye)
    return scores.reshape(_B)
